# baseline (speedup 1.0000x reference)
import functools

import jax
import jax.numpy as jnp
from jax import lax
from jax.experimental import pallas as pl
from jax.experimental.pallas import tpu as pltpu
from jax.experimental.pallas import tpu_sc as plsc

BM = 256
SC_W = 64
SC_NW = 32


def _gating_kernel(x_ref, wg_ref, ts_ref, probs_ref, sel_ref, cabs_ref,
                   counts_ref, carry_ref, *, n_experts, top_k, n_blocks):
    tb = pl.program_id(0)

    @pl.when(tb == 0)
    def _init():
        carry_ref[...] = jnp.zeros_like(carry_ref)

    x = x_ref[...]
    g = jax.lax.dot_general(x, wg_ref[...], (((1,), (1,)), ((), ())),
                            preferred_element_type=jnp.float32)
    twg = g * jax.nn.sigmoid(ts_ref[...])
    bm = twg.shape[0]
    lane_iota = jax.lax.broadcasted_iota(jnp.int32, (bm, n_experts), 1)
    work = twg
    vals, idxs, onehots = [], [], []
    for _ in range(top_k):
        m = jnp.max(work, axis=-1, keepdims=True)
        idx = jnp.argmax(work, axis=-1)
        oh = (lane_iota == idx[:, None]).astype(jnp.float32)
        vals.append(m)
        idxs.append(idx.astype(jnp.int32)[:, None])
        onehots.append(oh)
        work = jnp.where(oh > 0, -jnp.inf, work)
    v = jnp.concatenate(vals, axis=-1)
    v = v - v[:, 0:1]
    p = jnp.exp(v)
    p = p / jnp.sum(p, axis=-1, keepdims=True)
    probs_ref[...] = p
    sel_ref[...] = jnp.concatenate(idxs, axis=-1)

    mask = onehots[0]
    for k in range(1, top_k):
        mask = mask + onehots[k]
    r_iota = jax.lax.broadcasted_iota(jnp.int32, (bm, bm), 0)
    c_iota = jax.lax.broadcasted_iota(jnp.int32, (bm, bm), 1)
    tril = (r_iota >= c_iota).astype(jnp.bfloat16)
    cl = jnp.dot(tril, mask.astype(jnp.bfloat16),
                 preferred_element_type=jnp.float32)
    carry = carry_ref[...]
    cabs_ref[...] = cl + carry
    new_carry = carry + cl[bm - 1:bm, :]
    carry_ref[...] = new_carry

    @pl.when(tb == n_blocks - 1)
    def _emit_counts():
        counts_ref[...] = new_carry


def _index_kernel(counts_ref, cabs_ref, sel_ref, pos_ref, be_ref, bpad_ref,
                  *, n_experts, top_k, n_mm_blocks):
    cnt = counts_ref[...]
    padded = jnp.floor((cnt + (BM - 1)) * (1.0 / BM)) * BM
    pad_col = jnp.transpose(padded)
    r_iota = jax.lax.broadcasted_iota(jnp.int32, (n_experts, n_experts), 0)
    c_iota = jax.lax.broadcasted_iota(jnp.int32, (n_experts, n_experts), 1)
    upper = (r_iota < c_iota).astype(jnp.float32)
    offs = jnp.sum(pad_col * upper, axis=0, keepdims=True)

    cabs = cabs_ref[...]
    sel = sel_ref[...]
    bm = cabs.shape[0]
    lane_iota = jax.lax.broadcasted_iota(jnp.int32, (bm, n_experts), 1)
    pos_cols = []
    for k in range(top_k):
        oh = (lane_iota == sel[:, k:k + 1]).astype(jnp.float32)
        c_sel = jnp.sum(cabs * oh, axis=1, keepdims=True)
        off_sel = jnp.sum(offs * oh, axis=1, keepdims=True)
        pos_cols.append(off_sel + c_sel - 1.0)
    pos_blk = jnp.concatenate(pos_cols, axis=-1)
    pos_ref[...] = jnp.transpose(pos_blk).astype(jnp.int32)

    cumpad_col = jnp.transpose(offs + padded)
    bstart = (jax.lax.broadcasted_iota(
        jnp.int32, (1, n_mm_blocks), 1).astype(jnp.float32) * float(BM))
    be = jnp.sum((cumpad_col <= bstart).astype(jnp.float32), axis=0,
                 keepdims=True)
    be = jnp.minimum(be, float(n_experts - 1))
    sub_iota = jax.lax.broadcasted_iota(
        jnp.int32, (n_experts, n_mm_blocks), 0).astype(jnp.float32)
    ohb = (sub_iota == be).astype(jnp.float32)
    valid_end = jnp.sum(ohb * (jnp.transpose(offs) + jnp.transpose(cnt)),
                        axis=0, keepdims=True)
    be_ref[...] = be.astype(jnp.int32)
    bpad_ref[...] = (bstart >= valid_end).astype(jnp.int32)


def _sc_dispatch(x_flat, pos2d, sp_rows):
    T, D = x_flat.shape
    n_chunks = pos2d.shape[0]
    cpw = n_chunks // SC_NW
    mesh = plsc.VectorSubcoreMesh(core_axis_name="c", subcore_axis_name="s")

    @functools.partial(
        pl.kernel,
        out_type=jax.ShapeDtypeStruct((sp_rows, D), x_flat.dtype),
        mesh=mesh,
        scratch_types=[
            pltpu.VMEM((1, SC_W), jnp.int32),
            pltpu.VMEM((SC_W, D), x_flat.dtype),
            pltpu.SemaphoreType.DMA,
        ],
    )
    def k(x_hbm, pos_hbm, xs_hbm, pos_v, rows_v, sem):
        wid = lax.axis_index("s") * 2 + lax.axis_index("c")

        @pl.loop(0, cpw)
        def _(c):
            r = wid * cpw + c
            t0 = lax.rem(r, T // SC_W) * SC_W
            pltpu.sync_copy(x_hbm.at[pl.ds(t0, SC_W)], rows_v)
            pltpu.sync_copy(pos_hbm.at[pl.ds(r, 1)], pos_v)
            pltpu.async_copy(rows_v, xs_hbm.at[pos_v.at[0]], sem).wait()

    return k(x_flat, pos2d)


def _sc_combine_gather(eo_sorted, pos2d, n_pairs):
    D = eo_sorted.shape[1]
    n_chunks = pos2d.shape[0]
    cpw = n_chunks // SC_NW
    mesh = plsc.VectorSubcoreMesh(core_axis_name="c", subcore_axis_name="s")

    @functools.partial(
        pl.kernel,
        out_type=jax.ShapeDtypeStruct((n_pairs, D), eo_sorted.dtype),
        mesh=mesh,
        scratch_types=[
            pltpu.VMEM((1, SC_W), jnp.int32),
            pltpu.VMEM((SC_W, D), eo_sorted.dtype),
            pltpu.SemaphoreType.DMA,
        ],
    )
    def k(eo_hbm, pos_hbm, eop_hbm, pos_v, rows_v, sem):
        wid = lax.axis_index("s") * 2 + lax.axis_index("c")

        @pl.loop(0, cpw)
        def _(c):
            r = wid * cpw + c
            pltpu.sync_copy(pos_hbm.at[pl.ds(r, 1)], pos_v)
            pltpu.async_copy(eo_hbm.at[pos_v.at[0]], rows_v, sem).wait()
            pltpu.sync_copy(rows_v, eop_hbm.at[pl.ds(r * SC_W, SC_W)])

    return k(eo_sorted, pos2d)


def _group_mm_kernel(be_ref, bpad_ref, xs_ref, w1_ref, b1_ref, w2_ref,
                     b2_ref, out_ref):
    b = pl.program_id(0)

    @pl.when(bpad_ref[b] == 0)
    def _compute():
        x = xs_ref[...]
        w1 = w1_ref[0]
        h = jax.lax.dot_general(x, w1, (((1,), (1,)), ((), ())),
                                preferred_element_type=jnp.float32)
        h = h + b1_ref[0]
        h = 0.5 * h * (1.0 + jax.lax.erf(h * 0.7071067811865476))
        w2 = w2_ref[0]
        eo = jax.lax.dot_general(
            h.astype(jnp.bfloat16), w2, (((1,), (1,)), ((), ())),
            preferred_element_type=jnp.float32) + b2_ref[0]
        out_ref[...] = eo.astype(jnp.bfloat16)


def _combine_kernel(probs_ref, e0_ref, e1_ref, e2_ref, e3_ref, out_ref):
    p = probs_ref[...]
    out = p[:, 0:1] * e0_ref[...].astype(jnp.float32)
    out += p[:, 1:2] * e1_ref[...].astype(jnp.float32)
    out += p[:, 2:3] * e2_ref[...].astype(jnp.float32)
    out += p[:, 3:4] * e3_ref[...].astype(jnp.float32)
    out_ref[...] = out


def kernel(x, Wg, W1, b1, W2, b2, trust_scores):
    Bq, Sq, Dq = x.shape
    x_flat = x.reshape(-1, Dq)
    T = x_flat.shape[0]
    E, H, D = W1.shape
    K = 4
    n_tb = T // BM
    n_pairs = T * K
    NBLK = n_pairs // BM + E
    SP = NBLK * BM

    probs, sel, cabs, counts = pl.pallas_call(
        functools.partial(_gating_kernel, n_experts=E, top_k=K,
                          n_blocks=n_tb),
        grid=(n_tb,),
        in_specs=[
            pl.BlockSpec((BM, D), lambda i: (i, 0)),
            pl.BlockSpec((E, D), lambda i: (0, 0)),
            pl.BlockSpec((1, E), lambda i: (0, 0)),
        ],
        out_specs=[
            pl.BlockSpec((BM, K), lambda i: (i, 0)),
            pl.BlockSpec((BM, K), lambda i: (i, 0)),
            pl.BlockSpec((BM, E), lambda i: (i, 0)),
            pl.BlockSpec((1, E), lambda i: (0, 0)),
        ],
        out_shape=[
            jax.ShapeDtypeStruct((T, K), jnp.float32),
            jax.ShapeDtypeStruct((T, K), jnp.int32),
            jax.ShapeDtypeStruct((T, E), jnp.float32),
            jax.ShapeDtypeStruct((1, E), jnp.float32),
        ],
        scratch_shapes=[pltpu.VMEM((1, E), jnp.float32)],
        compiler_params=pltpu.CompilerParams(
            dimension_semantics=("arbitrary",),
        ),
    )(x_flat, Wg, trust_scores.reshape(1, E))

    pos_kt, be, bpad = pl.pallas_call(
        functools.partial(_index_kernel, n_experts=E, top_k=K,
                          n_mm_blocks=NBLK),
        grid=(n_tb,),
        in_specs=[
            pl.BlockSpec((1, E), lambda i: (0, 0)),
            pl.BlockSpec((BM, E), lambda i: (i, 0)),
            pl.BlockSpec((BM, K), lambda i: (i, 0)),
        ],
        out_specs=[
            pl.BlockSpec((K, BM), lambda i: (0, i)),
            pl.BlockSpec((1, NBLK), lambda i: (0, 0)),
            pl.BlockSpec((1, NBLK), lambda i: (0, 0)),
        ],
        out_shape=[
            jax.ShapeDtypeStruct((K, T), jnp.int32),
            jax.ShapeDtypeStruct((1, NBLK), jnp.int32),
            jax.ShapeDtypeStruct((1, NBLK), jnp.int32),
        ],
    )(counts, cabs, sel)

    pos2d = pos_kt.reshape(n_pairs // SC_W, SC_W)

    x_bf = x_flat.astype(jnp.bfloat16)
    x_i32 = jax.lax.bitcast_convert_type(
        x_bf.reshape(T, D // 2, 2), jnp.int32)
    xs_i32 = _sc_dispatch(x_i32, pos2d, SP)
    x_sorted = jax.lax.bitcast_convert_type(
        xs_i32, jnp.bfloat16).reshape(SP, D)

    eo_sorted = pl.pallas_call(
        _group_mm_kernel,
        grid_spec=pltpu.PrefetchScalarGridSpec(
            num_scalar_prefetch=2,
            grid=(NBLK,),
            in_specs=[
                pl.BlockSpec((BM, D), lambda b, be, bp: (b, 0)),
                pl.BlockSpec((1, H, D), lambda b, be, bp: (be[b], 0, 0)),
                pl.BlockSpec((1, 1, H), lambda b, be, bp: (be[b], 0, 0)),
                pl.BlockSpec((1, D, H), lambda b, be, bp: (be[b], 0, 0)),
                pl.BlockSpec((1, 1, D), lambda b, be, bp: (be[b], 0, 0)),
            ],
            out_specs=pl.BlockSpec((BM, D), lambda b, be, bp: (b, 0)),
        ),
        out_shape=jax.ShapeDtypeStruct((SP, D), jnp.bfloat16),
        compiler_params=pltpu.CompilerParams(
            dimension_semantics=("arbitrary",),
        ),
    )(be.reshape(NBLK), bpad.reshape(NBLK), x_sorted, W1.astype(jnp.bfloat16),
      b1.reshape(E, 1, H), W2.astype(jnp.bfloat16), b2.reshape(E, 1, D))

    eo_i32 = jax.lax.bitcast_convert_type(
        eo_sorted.reshape(SP, D // 2, 2), jnp.int32)
    eop_i32 = _sc_combine_gather(eo_i32, pos2d, n_pairs)
    eo_pairs = jax.lax.bitcast_convert_type(
        eop_i32, jnp.bfloat16).reshape(n_pairs, D)

    out = pl.pallas_call(
        _combine_kernel,
        grid=(n_tb,),
        in_specs=[
            pl.BlockSpec((BM, K), lambda i: (i, 0)),
            pl.BlockSpec((BM, D), lambda i: (0 * n_tb + i, 0)),
            pl.BlockSpec((BM, D), lambda i: (1 * n_tb + i, 0)),
            pl.BlockSpec((BM, D), lambda i: (2 * n_tb + i, 0)),
            pl.BlockSpec((BM, D), lambda i: (3 * n_tb + i, 0)),
        ],
        out_specs=pl.BlockSpec((BM, D), lambda i: (i, 0)),
        out_shape=jax.ShapeDtypeStruct((T, D), jnp.float32),
    )(probs, eo_pairs, eo_pairs, eo_pairs, eo_pairs)

    return out.reshape(Bq, Sq, Dq)


# speedup vs baseline: 2.2085x; 2.2085x over previous
import functools

import jax
import jax.numpy as jnp
from jax import lax
from jax.experimental import pallas as pl
from jax.experimental.pallas import tpu as pltpu
from jax.experimental.pallas import tpu_sc as plsc

BM = 256
SC_W = 64
SC_NW = 32


def _gating_kernel(x_ref, wg_ref, ts_ref, probs_ref, sel_ref, cabs_ref,
                   counts_ref, carry_ref, *, n_experts, top_k, n_blocks):
    tb = pl.program_id(0)

    @pl.when(tb == 0)
    def _init():
        carry_ref[...] = jnp.zeros_like(carry_ref)

    x = x_ref[...]
    g = jax.lax.dot_general(x, wg_ref[...], (((1,), (1,)), ((), ())),
                            preferred_element_type=jnp.float32)
    twg = g * jax.nn.sigmoid(ts_ref[...])
    bm = twg.shape[0]
    lane_iota = jax.lax.broadcasted_iota(jnp.int32, (bm, n_experts), 1)
    work = twg
    vals, idxs, onehots = [], [], []
    for _ in range(top_k):
        m = jnp.max(work, axis=-1, keepdims=True)
        idx = jnp.argmax(work, axis=-1)
        oh = (lane_iota == idx[:, None]).astype(jnp.float32)
        vals.append(m)
        idxs.append(idx.astype(jnp.int32)[:, None])
        onehots.append(oh)
        work = jnp.where(oh > 0, -jnp.inf, work)
    v = jnp.concatenate(vals, axis=-1)
    v = v - v[:, 0:1]
    p = jnp.exp(v)
    p = p / jnp.sum(p, axis=-1, keepdims=True)
    probs_ref[...] = p
    sel_ref[...] = jnp.concatenate(idxs, axis=-1)

    mask = onehots[0]
    for k in range(1, top_k):
        mask = mask + onehots[k]
    r_iota = jax.lax.broadcasted_iota(jnp.int32, (bm, bm), 0)
    c_iota = jax.lax.broadcasted_iota(jnp.int32, (bm, bm), 1)
    tril = (r_iota >= c_iota).astype(jnp.bfloat16)
    cl = jnp.dot(tril, mask.astype(jnp.bfloat16),
                 preferred_element_type=jnp.float32)
    carry = carry_ref[...]
    cabs_ref[...] = cl + carry
    new_carry = carry + cl[bm - 1:bm, :]
    carry_ref[...] = new_carry

    @pl.when(tb == n_blocks - 1)
    def _emit_counts():
        counts_ref[...] = new_carry


def _index_kernel(counts_ref, cabs_ref, sel_ref, pos_ref, be_ref, bpad_ref,
                  *, n_experts, top_k, n_mm_blocks):
    cnt = counts_ref[...]
    padded = jnp.floor((cnt + (BM - 1)) * (1.0 / BM)) * BM
    pad_col = jnp.transpose(padded)
    r_iota = jax.lax.broadcasted_iota(jnp.int32, (n_experts, n_experts), 0)
    c_iota = jax.lax.broadcasted_iota(jnp.int32, (n_experts, n_experts), 1)
    upper = (r_iota < c_iota).astype(jnp.float32)
    offs = jnp.sum(pad_col * upper, axis=0, keepdims=True)

    cabs = cabs_ref[...]
    sel = sel_ref[...]
    bm = cabs.shape[0]
    lane_iota = jax.lax.broadcasted_iota(jnp.int32, (bm, n_experts), 1)
    pos_cols = []
    for k in range(top_k):
        oh = (lane_iota == sel[:, k:k + 1]).astype(jnp.float32)
        c_sel = jnp.sum(cabs * oh, axis=1, keepdims=True)
        off_sel = jnp.sum(offs * oh, axis=1, keepdims=True)
        pos_cols.append(off_sel + c_sel - 1.0)
    pos_blk = jnp.concatenate(pos_cols, axis=-1)
    pos_ref[...] = jnp.transpose(pos_blk).astype(jnp.int32)

    cumpad_col = jnp.transpose(offs + padded)
    bstart = (jax.lax.broadcasted_iota(
        jnp.int32, (1, n_mm_blocks), 1).astype(jnp.float32) * float(BM))
    be = jnp.sum((cumpad_col <= bstart).astype(jnp.float32), axis=0,
                 keepdims=True)
    be = jnp.minimum(be, float(n_experts - 1))
    sub_iota = jax.lax.broadcasted_iota(
        jnp.int32, (n_experts, n_mm_blocks), 0).astype(jnp.float32)
    ohb = (sub_iota == be).astype(jnp.float32)
    valid_end = jnp.sum(ohb * (jnp.transpose(offs) + jnp.transpose(cnt)),
                        axis=0, keepdims=True)
    be_ref[...] = be.astype(jnp.int32)
    bpad_ref[...] = (bstart >= valid_end).astype(jnp.int32)


def _sc_dispatch(x_flat, pos2d, sp_rows):
    T, D = x_flat.shape
    n_chunks = pos2d.shape[0]
    cpw = n_chunks // SC_NW
    mesh = plsc.VectorSubcoreMesh(core_axis_name="c", subcore_axis_name="s")

    @functools.partial(
        pl.kernel,
        out_type=jax.ShapeDtypeStruct((sp_rows, D), x_flat.dtype),
        mesh=mesh,
        scratch_types=[
            pltpu.VMEM((1, SC_W), jnp.int32),
            pltpu.VMEM((SC_W, D), x_flat.dtype),
            pltpu.SemaphoreType.DMA,
        ],
    )
    def k(x_hbm, pos_hbm, xs_hbm, pos_v, rows_v, sem):
        wid = lax.axis_index("s") * 2 + lax.axis_index("c")

        @pl.loop(0, cpw)
        def _(c):
            r = wid * cpw + c
            t0 = lax.rem(r, T // SC_W) * SC_W
            pltpu.sync_copy(x_hbm.at[pl.ds(t0, SC_W)], rows_v)
            pltpu.sync_copy(pos_hbm.at[pl.ds(r, 1)], pos_v)
            pltpu.async_copy(rows_v, xs_hbm.at[pos_v.at[0]], sem).wait()

    return k(x_flat, pos2d)


def _sc_combine_gather(eo_sorted, pos2d, n_pairs):
    D = eo_sorted.shape[1]
    n_chunks = pos2d.shape[0]
    cpw = n_chunks // SC_NW
    mesh = plsc.VectorSubcoreMesh(core_axis_name="c", subcore_axis_name="s")

    @functools.partial(
        pl.kernel,
        out_type=jax.ShapeDtypeStruct((n_pairs, D), eo_sorted.dtype),
        mesh=mesh,
        scratch_types=[
            pltpu.VMEM((1, SC_W), jnp.int32),
            pltpu.VMEM((SC_W, D), eo_sorted.dtype),
            pltpu.SemaphoreType.DMA,
        ],
    )
    def k(eo_hbm, pos_hbm, eop_hbm, pos_v, rows_v, sem):
        wid = lax.axis_index("s") * 2 + lax.axis_index("c")

        @pl.loop(0, cpw)
        def _(c):
            r = wid * cpw + c
            pltpu.sync_copy(pos_hbm.at[pl.ds(r, 1)], pos_v)
            pltpu.async_copy(eo_hbm.at[pos_v.at[0]], rows_v, sem).wait()
            pltpu.sync_copy(rows_v, eop_hbm.at[pl.ds(r * SC_W, SC_W)])

    return k(eo_sorted, pos2d)


def _group_mm_kernel(be_ref, bpad_ref, xs_ref, w1_ref, b1_ref, w2_ref,
                     b2_ref, out_ref):
    b = pl.program_id(0)

    @pl.when(bpad_ref[b] == 0)
    def _compute():
        x = xs_ref[...].astype(jnp.bfloat16)
        w1 = w1_ref[0]
        h = jax.lax.dot_general(x, w1, (((1,), (1,)), ((), ())),
                                preferred_element_type=jnp.float32)
        h = h + b1_ref[0]
        h = 0.5 * h * (1.0 + jax.lax.erf(h * 0.7071067811865476))
        w2 = w2_ref[0]
        out_ref[...] = jax.lax.dot_general(
            h.astype(jnp.bfloat16), w2, (((1,), (1,)), ((), ())),
            preferred_element_type=jnp.float32) + b2_ref[0]


def _combine_kernel(probs_ref, e0_ref, e1_ref, e2_ref, e3_ref, out_ref):
    p = probs_ref[...]
    out = p[:, 0:1] * e0_ref[...]
    out += p[:, 1:2] * e1_ref[...]
    out += p[:, 2:3] * e2_ref[...]
    out += p[:, 3:4] * e3_ref[...]
    out_ref[...] = out


def kernel(x, Wg, W1, b1, W2, b2, trust_scores):
    Bq, Sq, Dq = x.shape
    x_flat = x.reshape(-1, Dq)
    T = x_flat.shape[0]
    E, H, D = W1.shape
    K = 4
    n_tb = T // BM
    n_pairs = T * K
    NBLK = n_pairs // BM + E
    SP = NBLK * BM

    probs, sel, cabs, counts = pl.pallas_call(
        functools.partial(_gating_kernel, n_experts=E, top_k=K,
                          n_blocks=n_tb),
        grid=(n_tb,),
        in_specs=[
            pl.BlockSpec((BM, D), lambda i: (i, 0)),
            pl.BlockSpec((E, D), lambda i: (0, 0)),
            pl.BlockSpec((1, E), lambda i: (0, 0)),
        ],
        out_specs=[
            pl.BlockSpec((BM, K), lambda i: (i, 0)),
            pl.BlockSpec((BM, K), lambda i: (i, 0)),
            pl.BlockSpec((BM, E), lambda i: (i, 0)),
            pl.BlockSpec((1, E), lambda i: (0, 0)),
        ],
        out_shape=[
            jax.ShapeDtypeStruct((T, K), jnp.float32),
            jax.ShapeDtypeStruct((T, K), jnp.int32),
            jax.ShapeDtypeStruct((T, E), jnp.float32),
            jax.ShapeDtypeStruct((1, E), jnp.float32),
        ],
        scratch_shapes=[pltpu.VMEM((1, E), jnp.float32)],
        compiler_params=pltpu.CompilerParams(
            dimension_semantics=("arbitrary",),
        ),
    )(x_flat, Wg, trust_scores.reshape(1, E))

    pos_kt, be, bpad = pl.pallas_call(
        functools.partial(_index_kernel, n_experts=E, top_k=K,
                          n_mm_blocks=NBLK),
        grid=(n_tb,),
        in_specs=[
            pl.BlockSpec((1, E), lambda i: (0, 0)),
            pl.BlockSpec((BM, E), lambda i: (i, 0)),
            pl.BlockSpec((BM, K), lambda i: (i, 0)),
        ],
        out_specs=[
            pl.BlockSpec((K, BM), lambda i: (0, i)),
            pl.BlockSpec((1, NBLK), lambda i: (0, 0)),
            pl.BlockSpec((1, NBLK), lambda i: (0, 0)),
        ],
        out_shape=[
            jax.ShapeDtypeStruct((K, T), jnp.int32),
            jax.ShapeDtypeStruct((1, NBLK), jnp.int32),
            jax.ShapeDtypeStruct((1, NBLK), jnp.int32),
        ],
    )(counts, cabs, sel)

    pos2d = pos_kt.reshape(n_pairs // SC_W, SC_W)

    x_sorted = _sc_dispatch(x_flat, pos2d, SP)

    eo_sorted = pl.pallas_call(
        _group_mm_kernel,
        grid_spec=pltpu.PrefetchScalarGridSpec(
            num_scalar_prefetch=2,
            grid=(NBLK,),
            in_specs=[
                pl.BlockSpec((BM, D), lambda b, be, bp: (b, 0)),
                pl.BlockSpec((1, H, D), lambda b, be, bp: (be[b], 0, 0)),
                pl.BlockSpec((1, 1, H), lambda b, be, bp: (be[b], 0, 0)),
                pl.BlockSpec((1, D, H), lambda b, be, bp: (be[b], 0, 0)),
                pl.BlockSpec((1, 1, D), lambda b, be, bp: (be[b], 0, 0)),
            ],
            out_specs=pl.BlockSpec((BM, D), lambda b, be, bp: (b, 0)),
        ),
        out_shape=jax.ShapeDtypeStruct((SP, D), jnp.float32),
        compiler_params=pltpu.CompilerParams(
            dimension_semantics=("arbitrary",),
        ),
    )(be.reshape(NBLK), bpad.reshape(NBLK), x_sorted, W1.astype(jnp.bfloat16),
      b1.reshape(E, 1, H), W2.astype(jnp.bfloat16), b2.reshape(E, 1, D))

    eo_pairs = _sc_combine_gather(eo_sorted, pos2d, n_pairs)

    out = pl.pallas_call(
        _combine_kernel,
        grid=(n_tb,),
        in_specs=[
            pl.BlockSpec((BM, K), lambda i: (i, 0)),
            pl.BlockSpec((BM, D), lambda i: (0 * n_tb + i, 0)),
            pl.BlockSpec((BM, D), lambda i: (1 * n_tb + i, 0)),
            pl.BlockSpec((BM, D), lambda i: (2 * n_tb + i, 0)),
            pl.BlockSpec((BM, D), lambda i: (3 * n_tb + i, 0)),
        ],
        out_specs=pl.BlockSpec((BM, D), lambda i: (i, 0)),
        out_shape=jax.ShapeDtypeStruct((T, D), jnp.float32),
    )(probs, eo_pairs, eo_pairs, eo_pairs, eo_pairs)

    return out.reshape(Bq, Sq, Dq)


# speedup vs baseline: 2.2308x; 1.0101x over previous
import functools

import jax
import jax.numpy as jnp
from jax import lax
from jax.experimental import pallas as pl
from jax.experimental.pallas import tpu as pltpu
from jax.experimental.pallas import tpu_sc as plsc

BM = 256
SC_W = 64
SC_NW = 32


def _gating_kernel(x_ref, wg_ref, ts_ref, probs_ref, pos_ref, be_ref,
                   bpad_ref, carry_ref, cabs_ref, sel_ref,
                   *, n_experts, top_k, n_blocks, n_mm_blocks):
    tb = pl.program_id(0)

    @pl.when(tb == 0)
    def _init():
        carry_ref[...] = jnp.zeros_like(carry_ref)

    @pl.when(tb < n_blocks)
    def _gate():
        x = x_ref[...]
        g = jax.lax.dot_general(x, wg_ref[...], (((1,), (1,)), ((), ())),
                                preferred_element_type=jnp.float32)
        twg = g * jax.nn.sigmoid(ts_ref[...])
        bm = twg.shape[0]
        lane_iota = jax.lax.broadcasted_iota(jnp.int32, (bm, n_experts), 1)
        work = twg
        vals, idxs, onehots = [], [], []
        for _ in range(top_k):
            m = jnp.max(work, axis=-1, keepdims=True)
            idx = jnp.argmax(work, axis=-1)
            oh = (lane_iota == idx[:, None]).astype(jnp.float32)
            vals.append(m)
            idxs.append(idx.astype(jnp.int32)[:, None])
            onehots.append(oh)
            work = jnp.where(oh > 0, -jnp.inf, work)
        v = jnp.concatenate(vals, axis=-1)
        v = v - v[:, 0:1]
        p = jnp.exp(v)
        p = p / jnp.sum(p, axis=-1, keepdims=True)
        probs_ref[...] = p
        rows = pl.ds(tb * BM, BM)
        sel_ref[rows, :] = jnp.concatenate(idxs, axis=-1)

        mask = onehots[0]
        for k in range(1, top_k):
            mask = mask + onehots[k]
        r_iota = jax.lax.broadcasted_iota(jnp.int32, (bm, bm), 0)
        c_iota = jax.lax.broadcasted_iota(jnp.int32, (bm, bm), 1)
        tril = (r_iota >= c_iota).astype(jnp.bfloat16)
        cl = jnp.dot(tril, mask.astype(jnp.bfloat16),
                     preferred_element_type=jnp.float32)
        carry = carry_ref[...]
        cabs_ref[rows, :] = cl + carry
        carry_ref[...] = carry + cl[bm - 1:bm, :]

    @pl.when(tb == n_blocks)
    def _index():
        cnt = carry_ref[...]
        padded = jnp.floor((cnt + (BM - 1)) * (1.0 / BM)) * BM
        pad_col = jnp.transpose(padded)
        r_iota = jax.lax.broadcasted_iota(jnp.int32, (n_experts, n_experts),
                                          0)
        c_iota = jax.lax.broadcasted_iota(jnp.int32, (n_experts, n_experts),
                                          1)
        upper = (r_iota < c_iota).astype(jnp.float32)
        offs = jnp.sum(pad_col * upper, axis=0, keepdims=True)

        cabs = cabs_ref[...]
        sel = sel_ref[...]
        t_all = cabs.shape[0]
        lane_iota = jax.lax.broadcasted_iota(jnp.int32, (t_all, n_experts), 1)
        pos_cols = []
        for k in range(top_k):
            oh = (lane_iota == sel[:, k:k + 1]).astype(jnp.float32)
            c_sel = jnp.sum(cabs * oh, axis=1, keepdims=True)
            off_sel = jnp.sum(offs * oh, axis=1, keepdims=True)
            pos_cols.append(off_sel + c_sel - 1.0)
        pos_blk = jnp.concatenate(pos_cols, axis=-1)
        pos_ref[...] = jnp.transpose(pos_blk).astype(jnp.int32)

        cumpad_col = jnp.transpose(offs + padded)
        bstart = (jax.lax.broadcasted_iota(
            jnp.int32, (1, n_mm_blocks), 1).astype(jnp.float32) * float(BM))
        be = jnp.sum((cumpad_col <= bstart).astype(jnp.float32), axis=0,
                     keepdims=True)
        be = jnp.minimum(be, float(n_experts - 1))
        sub_iota = jax.lax.broadcasted_iota(
            jnp.int32, (n_experts, n_mm_blocks), 0).astype(jnp.float32)
        ohb = (sub_iota == be).astype(jnp.float32)
        valid_end = jnp.sum(ohb * (jnp.transpose(offs) + jnp.transpose(cnt)),
                            axis=0, keepdims=True)
        be_ref[...] = be.astype(jnp.int32)
        bpad_ref[...] = (bstart >= valid_end).astype(jnp.int32)


def _sc_dispatch(x_flat, pos2d, sp_rows):
    T, D = x_flat.shape
    n_chunks = pos2d.shape[0]
    cpw = n_chunks // SC_NW
    mesh = plsc.VectorSubcoreMesh(core_axis_name="c", subcore_axis_name="s")

    @functools.partial(
        pl.kernel,
        out_type=jax.ShapeDtypeStruct((sp_rows, D), x_flat.dtype),
        mesh=mesh,
        scratch_types=[
            pltpu.VMEM((1, SC_W), jnp.int32),
            pltpu.VMEM((SC_W, D), x_flat.dtype),
            pltpu.SemaphoreType.DMA,
        ],
    )
    def k(x_hbm, pos_hbm, xs_hbm, pos_v, rows_v, sem):
        wid = lax.axis_index("s") * 2 + lax.axis_index("c")

        @pl.loop(0, cpw)
        def _(c):
            r = wid * cpw + c
            t0 = lax.rem(r, T // SC_W) * SC_W
            pltpu.sync_copy(x_hbm.at[pl.ds(t0, SC_W)], rows_v)
            pltpu.sync_copy(pos_hbm.at[pl.ds(r, 1)], pos_v)
            pltpu.async_copy(rows_v, xs_hbm.at[pos_v.at[0]], sem).wait()

    return k(x_flat, pos2d)


def _sc_combine_gather(eo_sorted, pos2d, n_pairs):
    D = eo_sorted.shape[1]
    n_chunks = pos2d.shape[0]
    cpw = n_chunks // SC_NW
    mesh = plsc.VectorSubcoreMesh(core_axis_name="c", subcore_axis_name="s")

    @functools.partial(
        pl.kernel,
        out_type=jax.ShapeDtypeStruct((n_pairs, D), eo_sorted.dtype),
        mesh=mesh,
        scratch_types=[
            pltpu.VMEM((1, SC_W), jnp.int32),
            pltpu.VMEM((SC_W, D), eo_sorted.dtype),
            pltpu.SemaphoreType.DMA,
        ],
    )
    def k(eo_hbm, pos_hbm, eop_hbm, pos_v, rows_v, sem):
        wid = lax.axis_index("s") * 2 + lax.axis_index("c")

        @pl.loop(0, cpw)
        def _(c):
            r = wid * cpw + c
            pltpu.sync_copy(pos_hbm.at[pl.ds(r, 1)], pos_v)
            pltpu.async_copy(eo_hbm.at[pos_v.at[0]], rows_v, sem).wait()
            pltpu.sync_copy(rows_v, eop_hbm.at[pl.ds(r * SC_W, SC_W)])

    return k(eo_sorted, pos2d)


def _group_mm_kernel(be_ref, bpad_ref, xs_ref, w1_ref, b1_ref, w2_ref,
                     b2_ref, out_ref):
    b = pl.program_id(0)

    @pl.when(bpad_ref[b] == 0)
    def _compute():
        x = xs_ref[...].astype(jnp.bfloat16)
        w1 = w1_ref[0]
        h = jax.lax.dot_general(x, w1, (((1,), (1,)), ((), ())),
                                preferred_element_type=jnp.float32)
        h = h + b1_ref[0]
        h = 0.5 * h * (1.0 + jax.lax.erf(h * 0.7071067811865476))
        w2 = w2_ref[0]
        out_ref[...] = jax.lax.dot_general(
            h.astype(jnp.bfloat16), w2, (((1,), (1,)), ((), ())),
            preferred_element_type=jnp.float32) + b2_ref[0]


def _combine_kernel(probs_ref, e0_ref, e1_ref, e2_ref, e3_ref, out_ref):
    p = probs_ref[...]
    out = p[:, 0:1] * e0_ref[...]
    out += p[:, 1:2] * e1_ref[...]
    out += p[:, 2:3] * e2_ref[...]
    out += p[:, 3:4] * e3_ref[...]
    out_ref[...] = out


def kernel(x, Wg, W1, b1, W2, b2, trust_scores):
    Bq, Sq, Dq = x.shape
    x_flat = x.reshape(-1, Dq)
    T = x_flat.shape[0]
    E, H, D = W1.shape
    K = 4
    n_tb = T // BM
    n_pairs = T * K
    NBLK = n_pairs // BM + E
    SP = NBLK * BM

    last_tb = n_tb - 1
    probs, pos_kt, be, bpad = pl.pallas_call(
        functools.partial(_gating_kernel, n_experts=E, top_k=K,
                          n_blocks=n_tb, n_mm_blocks=NBLK),
        grid=(n_tb + 1,),
        in_specs=[
            pl.BlockSpec((BM, D), lambda i: (jnp.minimum(i, last_tb), 0)),
            pl.BlockSpec((E, D), lambda i: (0, 0)),
            pl.BlockSpec((1, E), lambda i: (0, 0)),
        ],
        out_specs=[
            pl.BlockSpec((BM, K), lambda i: (jnp.minimum(i, last_tb), 0)),
            pl.BlockSpec((K, T), lambda i: (0, 0)),
            pl.BlockSpec((1, NBLK), lambda i: (0, 0)),
            pl.BlockSpec((1, NBLK), lambda i: (0, 0)),
        ],
        out_shape=[
            jax.ShapeDtypeStruct((T, K), jnp.float32),
            jax.ShapeDtypeStruct((K, T), jnp.int32),
            jax.ShapeDtypeStruct((1, NBLK), jnp.int32),
            jax.ShapeDtypeStruct((1, NBLK), jnp.int32),
        ],
        scratch_shapes=[
            pltpu.VMEM((1, E), jnp.float32),
            pltpu.VMEM((T, E), jnp.float32),
            pltpu.VMEM((T, K), jnp.int32),
        ],
        compiler_params=pltpu.CompilerParams(
            dimension_semantics=("arbitrary",),
        ),
    )(x_flat, Wg, trust_scores.reshape(1, E))

    pos2d = pos_kt.reshape(n_pairs // SC_W, SC_W)

    x_sorted = _sc_dispatch(x_flat, pos2d, SP)

    eo_sorted = pl.pallas_call(
        _group_mm_kernel,
        grid_spec=pltpu.PrefetchScalarGridSpec(
            num_scalar_prefetch=2,
            grid=(NBLK,),
            in_specs=[
                pl.BlockSpec((BM, D), lambda b, be, bp: (b, 0)),
                pl.BlockSpec((1, H, D), lambda b, be, bp: (be[b], 0, 0)),
                pl.BlockSpec((1, 1, H), lambda b, be, bp: (be[b], 0, 0)),
                pl.BlockSpec((1, D, H), lambda b, be, bp: (be[b], 0, 0)),
                pl.BlockSpec((1, 1, D), lambda b, be, bp: (be[b], 0, 0)),
            ],
            out_specs=pl.BlockSpec((BM, D), lambda b, be, bp: (b, 0)),
        ),
        out_shape=jax.ShapeDtypeStruct((SP, D), jnp.float32),
        compiler_params=pltpu.CompilerParams(
            dimension_semantics=("arbitrary",),
        ),
    )(be.reshape(NBLK), bpad.reshape(NBLK), x_sorted, W1.astype(jnp.bfloat16),
      b1.reshape(E, 1, H), W2.astype(jnp.bfloat16), b2.reshape(E, 1, D))

    eo_pairs = _sc_combine_gather(eo_sorted, pos2d, n_pairs)

    out = pl.pallas_call(
        _combine_kernel,
        grid=(n_tb,),
        in_specs=[
            pl.BlockSpec((BM, K), lambda i: (i, 0)),
            pl.BlockSpec((BM, D), lambda i: (0 * n_tb + i, 0)),
            pl.BlockSpec((BM, D), lambda i: (1 * n_tb + i, 0)),
            pl.BlockSpec((BM, D), lambda i: (2 * n_tb + i, 0)),
            pl.BlockSpec((BM, D), lambda i: (3 * n_tb + i, 0)),
        ],
        out_specs=pl.BlockSpec((BM, D), lambda i: (i, 0)),
        out_shape=jax.ShapeDtypeStruct((T, D), jnp.float32),
    )(probs, eo_pairs, eo_pairs, eo_pairs, eo_pairs)

    return out.reshape(Bq, Sq, Dq)


# speedup vs baseline: 2.6698x; 1.1968x over previous
import functools

import jax
import jax.numpy as jnp
from jax import lax
from jax.experimental import pallas as pl
from jax.experimental.pallas import tpu as pltpu
from jax.experimental.pallas import tpu_sc as plsc

BM = 256
SC_W = 64
SC_NW = 32


def _gating_kernel(x_ref, wg_ref, ts_ref, probs_ref, pos_ref, be_ref,
                   bpad_ref, carry_ref, cabs_ref, sel_ref,
                   *, n_experts, top_k, n_blocks, n_mm_blocks):
    tb = pl.program_id(0)

    @pl.when(tb == 0)
    def _init():
        carry_ref[...] = jnp.zeros_like(carry_ref)

    @pl.when(tb < n_blocks)
    def _gate():
        x = x_ref[...]
        g = jax.lax.dot_general(x, wg_ref[...], (((1,), (1,)), ((), ())),
                                preferred_element_type=jnp.float32)
        twg = g * jax.nn.sigmoid(ts_ref[...])
        bm = twg.shape[0]
        lane_iota = jax.lax.broadcasted_iota(jnp.int32, (bm, n_experts), 1)
        work = twg
        vals, idxs, onehots = [], [], []
        for _ in range(top_k):
            m = jnp.max(work, axis=-1, keepdims=True)
            idx = jnp.argmax(work, axis=-1)
            oh = (lane_iota == idx[:, None]).astype(jnp.float32)
            vals.append(m)
            idxs.append(idx.astype(jnp.int32)[:, None])
            onehots.append(oh)
            work = jnp.where(oh > 0, -jnp.inf, work)
        v = jnp.concatenate(vals, axis=-1)
        v = v - v[:, 0:1]
        p = jnp.exp(v)
        p = p / jnp.sum(p, axis=-1, keepdims=True)
        probs_ref[...] = p
        rows = pl.ds(tb * BM, BM)
        sel_ref[rows, :] = jnp.concatenate(idxs, axis=-1)

        mask = onehots[0]
        for k in range(1, top_k):
            mask = mask + onehots[k]
        r_iota = jax.lax.broadcasted_iota(jnp.int32, (bm, bm), 0)
        c_iota = jax.lax.broadcasted_iota(jnp.int32, (bm, bm), 1)
        tril = (r_iota >= c_iota).astype(jnp.bfloat16)
        cl = jnp.dot(tril, mask.astype(jnp.bfloat16),
                     preferred_element_type=jnp.float32)
        carry = carry_ref[...]
        cabs_ref[rows, :] = cl + carry
        carry_ref[...] = carry + cl[bm - 1:bm, :]

    @pl.when(tb == n_blocks)
    def _index():
        cnt = carry_ref[...]
        padded = jnp.floor((cnt + (BM - 1)) * (1.0 / BM)) * BM
        pad_col = jnp.transpose(padded)
        r_iota = jax.lax.broadcasted_iota(jnp.int32, (n_experts, n_experts),
                                          0)
        c_iota = jax.lax.broadcasted_iota(jnp.int32, (n_experts, n_experts),
                                          1)
        upper = (r_iota < c_iota).astype(jnp.float32)
        offs = jnp.sum(pad_col * upper, axis=0, keepdims=True)

        cabs = cabs_ref[...]
        sel = sel_ref[...]
        t_all = cabs.shape[0]
        lane_iota = jax.lax.broadcasted_iota(jnp.int32, (t_all, n_experts), 1)
        pos_cols = []
        for k in range(top_k):
            oh = (lane_iota == sel[:, k:k + 1]).astype(jnp.float32)
            c_sel = jnp.sum(cabs * oh, axis=1, keepdims=True)
            off_sel = jnp.sum(offs * oh, axis=1, keepdims=True)
            pos_cols.append(off_sel + c_sel - 1.0)
        pos_blk = jnp.concatenate(pos_cols, axis=-1)
        pos_ref[...] = jnp.transpose(pos_blk).astype(jnp.int32)

        cumpad_col = jnp.transpose(offs + padded)
        bstart = (jax.lax.broadcasted_iota(
            jnp.int32, (1, n_mm_blocks), 1).astype(jnp.float32) * float(BM))
        be = jnp.sum((cumpad_col <= bstart).astype(jnp.float32), axis=0,
                     keepdims=True)
        be = jnp.minimum(be, float(n_experts - 1))
        sub_iota = jax.lax.broadcasted_iota(
            jnp.int32, (n_experts, n_mm_blocks), 0).astype(jnp.float32)
        ohb = (sub_iota == be).astype(jnp.float32)
        valid_end = jnp.sum(ohb * (jnp.transpose(offs) + jnp.transpose(cnt)),
                            axis=0, keepdims=True)
        be_ref[...] = be.astype(jnp.int32)
        bpad_ref[...] = (bstart >= valid_end).astype(jnp.int32)


def _sc_dispatch(x_flat, pos2d, sp_rows):
    T, D = x_flat.shape
    n_chunks = pos2d.shape[0]
    cpw = n_chunks // SC_NW
    mesh = plsc.VectorSubcoreMesh(core_axis_name="c", subcore_axis_name="s")

    @functools.partial(
        pl.kernel,
        out_type=jax.ShapeDtypeStruct((sp_rows, D), x_flat.dtype),
        mesh=mesh,
        scratch_types=[
            pltpu.VMEM((1, SC_W), jnp.int32),
            pltpu.VMEM((SC_W, D), x_flat.dtype),
            pltpu.SemaphoreType.DMA,
        ],
    )
    def k(x_hbm, pos_hbm, xs_hbm, pos_v, rows_v, sem):
        wid = lax.axis_index("s") * 2 + lax.axis_index("c")

        @pl.loop(0, cpw)
        def _(c):
            r = wid * cpw + c
            t0 = lax.rem(r, T // SC_W) * SC_W
            pltpu.sync_copy(x_hbm.at[pl.ds(t0, SC_W)], rows_v)
            pltpu.sync_copy(pos_hbm.at[pl.ds(r, 1)], pos_v)
            pltpu.async_copy(rows_v, xs_hbm.at[pos_v.at[0]], sem).wait()

    return k(x_flat, pos2d)


def _sc_combine_gather(eo_sorted, pos2d, n_pairs):
    D = eo_sorted.shape[1]
    n_chunks = pos2d.shape[0]
    cpw = n_chunks // SC_NW
    mesh = plsc.VectorSubcoreMesh(core_axis_name="c", subcore_axis_name="s")

    @functools.partial(
        pl.kernel,
        out_type=jax.ShapeDtypeStruct((n_pairs, D), eo_sorted.dtype),
        mesh=mesh,
        scratch_types=[
            pltpu.VMEM((1, SC_W), jnp.int32),
            pltpu.VMEM((SC_W, D), eo_sorted.dtype),
            pltpu.SemaphoreType.DMA,
        ],
    )
    def k(eo_hbm, pos_hbm, eop_hbm, pos_v, rows_v, sem):
        wid = lax.axis_index("s") * 2 + lax.axis_index("c")

        @pl.loop(0, cpw)
        def _(c):
            r = wid * cpw + c
            pltpu.sync_copy(pos_hbm.at[pl.ds(r, 1)], pos_v)
            pltpu.async_copy(eo_hbm.at[pos_v.at[0]], rows_v, sem).wait()
            pltpu.sync_copy(rows_v, eop_hbm.at[pl.ds(r * SC_W, SC_W)])

    return k(eo_sorted, pos2d)


def _ffn_half(xs_ref, w1_ref, b1_ref, w2_ref):
    x = xs_ref[...].astype(jnp.bfloat16)
    w1 = w1_ref[0].astype(jnp.bfloat16)
    h = jax.lax.dot_general(x, w1, (((1,), (1,)), ((), ())),
                            preferred_element_type=jnp.float32)
    h = h + b1_ref[0]
    h = 0.5 * h * (1.0 + jax.lax.erf(h * 0.7071067811865476))
    w2 = w2_ref[0].astype(jnp.bfloat16)
    return jax.lax.dot_general(
        h.astype(jnp.bfloat16), w2, (((1,), (1,)), ((), ())),
        preferred_element_type=jnp.float32)


def _group_mm_a_kernel(be_ref, bpad_ref, xs_ref, w1_ref, b1_ref, w2_ref,
                       b2_ref, out_ref):
    b = pl.program_id(0)

    @pl.when(bpad_ref[b] == 0)
    def _compute():
        out_ref[...] = _ffn_half(xs_ref, w1_ref, b1_ref, w2_ref) + b2_ref[0]


def _group_mm_b_kernel(be_ref, bpad_ref, prev_ref, xs_ref, w1_ref, b1_ref,
                       w2_ref, out_ref):
    b = pl.program_id(0)

    @pl.when(bpad_ref[b] == 0)
    def _compute():
        out_ref[...] = prev_ref[...] + _ffn_half(xs_ref, w1_ref, b1_ref,
                                                 w2_ref)


def _combine_kernel(probs_ref, e0_ref, e1_ref, e2_ref, e3_ref, out_ref):
    p = probs_ref[...]
    out = p[:, 0:1] * e0_ref[...]
    out += p[:, 1:2] * e1_ref[...]
    out += p[:, 2:3] * e2_ref[...]
    out += p[:, 3:4] * e3_ref[...]
    out_ref[...] = out


def kernel(x, Wg, W1, b1, W2, b2, trust_scores):
    Bq, Sq, Dq = x.shape
    x_flat = x.reshape(-1, Dq)
    T = x_flat.shape[0]
    E, H, D = W1.shape
    K = 4
    n_tb = T // BM
    n_pairs = T * K
    NBLK = n_pairs // BM + E
    SP = NBLK * BM

    last_tb = n_tb - 1
    probs, pos_kt, be, bpad = pl.pallas_call(
        functools.partial(_gating_kernel, n_experts=E, top_k=K,
                          n_blocks=n_tb, n_mm_blocks=NBLK),
        grid=(n_tb + 1,),
        in_specs=[
            pl.BlockSpec((BM, D), lambda i: (jnp.minimum(i, last_tb), 0)),
            pl.BlockSpec((E, D), lambda i: (0, 0)),
            pl.BlockSpec((1, E), lambda i: (0, 0)),
        ],
        out_specs=[
            pl.BlockSpec((BM, K), lambda i: (jnp.minimum(i, last_tb), 0)),
            pl.BlockSpec((K, T), lambda i: (0, 0)),
            pl.BlockSpec((1, NBLK), lambda i: (0, 0)),
            pl.BlockSpec((1, NBLK), lambda i: (0, 0)),
        ],
        out_shape=[
            jax.ShapeDtypeStruct((T, K), jnp.float32),
            jax.ShapeDtypeStruct((K, T), jnp.int32),
            jax.ShapeDtypeStruct((1, NBLK), jnp.int32),
            jax.ShapeDtypeStruct((1, NBLK), jnp.int32),
        ],
        scratch_shapes=[
            pltpu.VMEM((1, E), jnp.float32),
            pltpu.VMEM((T, E), jnp.float32),
            pltpu.VMEM((T, K), jnp.int32),
        ],
        compiler_params=pltpu.CompilerParams(
            dimension_semantics=("arbitrary",),
        ),
    )(x_flat, Wg, trust_scores.reshape(1, E))

    pos2d = pos_kt.reshape(n_pairs // SC_W, SC_W)

    x_sorted = _sc_dispatch(x_flat, pos2d, SP)

    HB = H // 2
    b1r = b1.reshape(E, 1, H)
    eo_a = pl.pallas_call(
        _group_mm_a_kernel,
        grid_spec=pltpu.PrefetchScalarGridSpec(
            num_scalar_prefetch=2,
            grid=(NBLK,),
            in_specs=[
                pl.BlockSpec((BM, D), lambda b, be, bp: (b, 0)),
                pl.BlockSpec((1, HB, D), lambda b, be, bp: (be[b], 0, 0)),
                pl.BlockSpec((1, 1, HB), lambda b, be, bp: (be[b], 0, 0)),
                pl.BlockSpec((1, D, HB), lambda b, be, bp: (be[b], 0, 0)),
                pl.BlockSpec((1, 1, D), lambda b, be, bp: (be[b], 0, 0)),
            ],
            out_specs=pl.BlockSpec((BM, D), lambda b, be, bp: (b, 0)),
        ),
        out_shape=jax.ShapeDtypeStruct((SP, D), jnp.float32),
        compiler_params=pltpu.CompilerParams(
            dimension_semantics=("arbitrary",),
        ),
    )(be.reshape(NBLK), bpad.reshape(NBLK), x_sorted, W1, b1r, W2,
      b2.reshape(E, 1, D))

    eo_sorted = pl.pallas_call(
        _group_mm_b_kernel,
        grid_spec=pltpu.PrefetchScalarGridSpec(
            num_scalar_prefetch=2,
            grid=(NBLK,),
            in_specs=[
                pl.BlockSpec((BM, D), lambda b, be, bp: (b, 0)),
                pl.BlockSpec((BM, D), lambda b, be, bp: (b, 0)),
                pl.BlockSpec((1, HB, D), lambda b, be, bp: (be[b], 1, 0)),
                pl.BlockSpec((1, 1, HB), lambda b, be, bp: (be[b], 0, 1)),
                pl.BlockSpec((1, D, HB), lambda b, be, bp: (be[b], 0, 1)),
            ],
            out_specs=pl.BlockSpec((BM, D), lambda b, be, bp: (b, 0)),
        ),
        out_shape=jax.ShapeDtypeStruct((SP, D), jnp.float32),
        compiler_params=pltpu.CompilerParams(
            dimension_semantics=("arbitrary",),
        ),
    )(be.reshape(NBLK), bpad.reshape(NBLK), eo_a, x_sorted, W1, b1r, W2)

    eo_pairs = _sc_combine_gather(eo_sorted, pos2d, n_pairs)

    out = pl.pallas_call(
        _combine_kernel,
        grid=(n_tb,),
        in_specs=[
            pl.BlockSpec((BM, K), lambda i: (i, 0)),
            pl.BlockSpec((BM, D), lambda i: (0 * n_tb + i, 0)),
            pl.BlockSpec((BM, D), lambda i: (1 * n_tb + i, 0)),
            pl.BlockSpec((BM, D), lambda i: (2 * n_tb + i, 0)),
            pl.BlockSpec((BM, D), lambda i: (3 * n_tb + i, 0)),
        ],
        out_specs=pl.BlockSpec((BM, D), lambda i: (i, 0)),
        out_shape=jax.ShapeDtypeStruct((T, D), jnp.float32),
    )(probs, eo_pairs, eo_pairs, eo_pairs, eo_pairs)

    return out.reshape(Bq, Sq, Dq)
